# Initial kernel scaffold; baseline (speedup 1.0000x reference)
import jax
import jax.numpy as jnp
from jax.experimental import pallas as pl


def kernel(slot_states, type_emb, slot_type_ids):
    raise NotImplementedError("write your pallas kernel here")


# speedup vs baseline: 1.9644x; 1.9644x over previous
import jax
import jax.numpy as jnp
from jax.experimental import pallas as pl
from jax.experimental.pallas import tpu as pltpu


def _body(x_ref, emb_ref, ids_ref, copy_ref, typed_ref, feat_ref, pat_ref):
    S, D = pat_ref.shape
    T = 3

    @pl.when(pl.program_id(0) == 0)
    def _():
        t = ids_ref[...]
        p = jnp.broadcast_to(emb_ref[T - 1:T, :], (S, D))
        for tt in range(T - 2, -1, -1):
            p = jnp.where(t == tt, jnp.broadcast_to(emb_ref[tt:tt + 1, :], (S, D)), p)
        pat_ref[...] = p

    x = x_ref[0]
    p = pat_ref[...]
    copy_ref[0] = x
    typed_ref[0] = x + p
    feat_ref[0] = p


def kernel(slot_states, type_emb, slot_type_ids):
    B, S, D = slot_states.shape
    T = type_emb.shape[0]
    ids_col = slot_type_ids.astype(jnp.int32).reshape(S, 1)
    emb_pad = jnp.concatenate(
        [type_emb, jnp.zeros((8 - T, D), dtype=type_emb.dtype)], axis=0)

    copy, typed, feat = pl.pallas_call(
        _body,
        grid=(B,),
        in_specs=[
            pl.BlockSpec((1, S, D), lambda b: (b, 0, 0)),
            pl.BlockSpec((8, D), lambda b: (0, 0)),
            pl.BlockSpec((S, 1), lambda b: (0, 0)),
        ],
        out_specs=[
            pl.BlockSpec((1, S, D), lambda b: (b, 0, 0)),
            pl.BlockSpec((1, S, D), lambda b: (b, 0, 0)),
            pl.BlockSpec((1, S, D), lambda b: (b, 0, 0)),
        ],
        out_shape=[jax.ShapeDtypeStruct((B, S, D), jnp.float32)] * 3,
        scratch_shapes=[pltpu.VMEM((S, D), jnp.float32)],
    )(slot_states, emb_pad, ids_col)

    type_ids = jnp.broadcast_to(slot_type_ids[None, :], (B, S))
    slot_mask = jnp.ones((B, S), dtype=jnp.bool_)
    return (copy, typed, type_ids, feat, slot_mask)


# speedup vs baseline: 2.1282x; 1.0834x over previous
import jax
import jax.numpy as jnp
from jax.experimental import pallas as pl
from jax.experimental.pallas import tpu as pltpu

_B_BLK = 2


def _body(x_ref, emb_ref, ids_ref, copy_ref, typed_ref, feat_ref, pat_ref):
    R, C = pat_ref.shape
    T = 3

    @pl.when(pl.program_id(0) == 0)
    def _():
        t = ids_ref[...]
        p = jnp.broadcast_to(emb_ref[T - 1:T, :], (R, C))
        for tt in range(T - 2, -1, -1):
            p = jnp.where(t == tt, jnp.broadcast_to(emb_ref[tt:tt + 1, :], (R, C)), p)
        pat_ref[...] = p

    x = x_ref[...]
    p = pat_ref[...][None]
    copy_ref[...] = x
    typed_ref[...] = x + p
    feat_ref[...] = jnp.broadcast_to(p, x.shape)


def kernel(slot_states, type_emb, slot_type_ids):
    B, S, D = slot_states.shape
    T = type_emb.shape[0]
    C = 128
    R = S * D // C
    rep = C // D

    x2 = slot_states.reshape(B, R, C)
    ids2 = jnp.broadcast_to(
        slot_type_ids.astype(jnp.int32).reshape(R, rep, 1), (R, rep, D)
    ).reshape(R, C)
    emb2 = jnp.concatenate(
        [jnp.tile(type_emb, (1, rep)), jnp.zeros((8 - T, C), dtype=type_emb.dtype)],
        axis=0)

    copy, typed, feat = pl.pallas_call(
        _body,
        grid=(B // _B_BLK,),
        in_specs=[
            pl.BlockSpec((_B_BLK, R, C), lambda b: (b, 0, 0)),
            pl.BlockSpec((8, C), lambda b: (0, 0)),
            pl.BlockSpec((R, C), lambda b: (0, 0)),
        ],
        out_specs=[
            pl.BlockSpec((_B_BLK, R, C), lambda b: (b, 0, 0)),
            pl.BlockSpec((_B_BLK, R, C), lambda b: (b, 0, 0)),
            pl.BlockSpec((_B_BLK, R, C), lambda b: (b, 0, 0)),
        ],
        out_shape=[jax.ShapeDtypeStruct((B, R, C), jnp.float32)] * 3,
        scratch_shapes=[pltpu.VMEM((R, C), jnp.float32)],
    )(x2, emb2, ids2)

    type_ids = jnp.broadcast_to(slot_type_ids[None, :], (B, S))
    slot_mask = jnp.ones((B, S), dtype=jnp.bool_)
    return (copy.reshape(B, S, D), typed.reshape(B, S, D),
            type_ids, feat.reshape(B, S, D), slot_mask)
